# Initial kernel scaffold; baseline (speedup 1.0000x reference)
import jax
import jax.numpy as jnp
from jax.experimental import pallas as pl


def kernel(x, edge_index, W1, b1, W2, b2, W3, b3, W_fc, b_fc):
    raise NotImplementedError("write your pallas kernel here")


# speedup vs baseline: 2.1523x; 2.1523x over previous
import functools

import jax
import jax.numpy as jnp
from jax import lax
from jax.experimental import pallas as pl
from jax.experimental.pallas import tpu as pltpu
from jax.experimental.pallas import tpu_sc as plsc

_N = 10000
_E = 160000
_NS = 16
_NC = 2
_EPT = _E // _NS
_K = 128
_NB = 80
_EPAD = _NB * _K
_HN = 5000
_HACC = 5008
_PBUF = 10752
_B = 128
_NH = 10240
_HPT = _NH // _NS


def _mesh():
    return plsc.VectorSubcoreMesh(core_axis_name="c", subcore_axis_name="s")


def _deg_body(idx2d, out_hbm, didx_v, hist_v, red_v, sum_v, stage_sh):
    cid = lax.axis_index("c")
    sid = lax.axis_index("s")
    one16 = jnp.full((16,), 1.0, jnp.float32)
    zer16 = jnp.zeros((16,), jnp.float32)

    def _fill_zer(i, c):
        hist_v[pl.ds(i * 16, 16)] = zer16
        return c
    lax.fori_loop(0, _NH // 16, _fill_zer, 0)

    pltpu.sync_copy(idx2d.at[pl.ds(cid * (_NS * _NB) + sid * _NB, _NB)],
                    didx_v)

    def _hist(b, c):
        for j in range(_K // 16):
            idx16 = didx_v[b, pl.ds(j * 16, 16)]
            plsc.addupdate_scatter(hist_v, [idx16], one16)
        return c
    lax.fori_loop(0, _NB, _hist, 0)

    pltpu.sync_copy(hist_v, stage_sh.at[sid])
    plsc.subcore_barrier()

    pltpu.sync_copy(stage_sh.at[:, pl.ds(sid * _HPT, _HPT)], red_v)

    def _red(j, c):
        s = zer16
        for r in range(_NS):
            s = s + red_v[r, pl.ds(j * 16, 16)]
        sum_v[pl.ds(j * 16, 16)] = s
        return c
    lax.fori_loop(0, _HPT // 16, _red, 0)

    pltpu.sync_copy(sum_v, out_hbm.at[cid, 0, pl.ds(sid * _HPT, _HPT)])


def _deg_call(idx2d):
    f = functools.partial(
        pl.kernel, mesh=_mesh(),
        compiler_params=pltpu.CompilerParams(needs_layout_passes=False),
        out_type=jax.ShapeDtypeStruct((_NC, 1, _NH), jnp.float32),
        scratch_types=[
            pltpu.VMEM((_NB, _K), jnp.int32),
            pltpu.VMEM((_NH,), jnp.float32),
            pltpu.VMEM((_NS, _HPT), jnp.float32),
            pltpu.VMEM((_HPT,), jnp.float32),
            pltpu.VMEM_SHARED((_NS, _NH), jnp.float32),
        ],
    )(_deg_body)
    return f(idx2d)


def _seg_body(m0, m1, m2, m3, sidx2d, didx2d, zer_hbm,
              o0, o1, o2, o3,
              sidx_v, didx_v, sp_v, dp_v, r0, r1, acc_sh, sem0, sem1):
    cid = lax.axis_index("c")
    sid = lax.axis_index("s")
    zero16 = jnp.zeros((16,), jnp.int32)
    trash16 = jnp.full((16,), _HN, jnp.int32)

    pltpu.sync_copy(sidx2d.at[pl.ds(sid * _NB, _NB)], sidx_v)
    pltpu.sync_copy(didx2d.at[pl.ds(sid * _NB, _NB)], didx_v)

    def _fill(i, c):
        sp_v[pl.ds(i * 16, 16)] = zero16
        dp_v[pl.ds(i * 16, 16)] = trash16
        return c
    lax.fori_loop(0, _PBUF // 16, _fill, 0)

    lo = cid * _HN

    def _part(b, off):
        for j in range(_K // 16):
            d = didx_v[b, pl.ds(j * 16, 16)]
            sv = sidx_v[b, pl.ds(j * 16, 16)]
            dl = d - lo
            msk = jnp.logical_and(dl >= 0, dl < _HN)
            plsc.store_compressed(sp_v.at[pl.ds(off, 16)], sv, mask=msk)
            plsc.store_compressed(dp_v.at[pl.ds(off, 16)], dl, mask=msk)
            off = off + plsc.all_reduce_population_count(msk)[0]
        return off
    cnt = lax.fori_loop(0, _NB, _part, 0)
    nb = ((cnt + 2 * _B - 1) // (2 * _B)) * 2

    for ci, (mr, our) in enumerate(((m0, o0), (m1, o1), (m2, o2), (m3, o3))):
        pltpu.sync_copy(zer_hbm.at[pl.ds(0, 312)],
                        acc_sh.at[pl.ds(sid * 312, 312)])

        @pl.when(sid == _NS - 1)
        def _():
            pltpu.sync_copy(zer_hbm.at[pl.ds(0, 16)],
                            acc_sh.at[pl.ds(4992, 16)])
        plsc.subcore_barrier()

        def _gather(b, buf, sem, mr=mr):
            pltpu.async_copy(mr.at[sp_v.at[pl.ds(b * _B, _B)]], buf, sem)

        def _wait(buf, sem, mr=mr):
            pltpu.make_async_copy(mr.at[pl.ds(0, _B)], buf, sem).wait()

        @pl.when(nb > 0)
        def _():
            _gather(0, r0, sem0)
            _gather(1, r1, sem1)

        def _pair(p, c):
            b0 = 2 * p
            _wait(r0, sem0)
            pltpu.sync_copy(r0, acc_sh.at[dp_v.at[pl.ds(b0 * _B, _B)]],
                            add=True)

            @pl.when(b0 + 2 < nb)
            def _():
                _gather(b0 + 2, r0, sem0)
            _wait(r1, sem1)
            pltpu.sync_copy(r1, acc_sh.at[dp_v.at[pl.ds((b0 + 1) * _B, _B)]],
                            add=True)

            @pl.when(b0 + 3 < nb)
            def _():
                _gather(b0 + 3, r1, sem1)
            return c
        lax.fori_loop(0, nb // 2, _pair, 0)

        plsc.subcore_barrier()
        pltpu.sync_copy(acc_sh.at[pl.ds(sid * 312, 312)],
                        our.at[pl.ds(lo + sid * 312, 312)])

        @pl.when(sid == _NS - 1)
        def _():
            pltpu.sync_copy(acc_sh.at[pl.ds(4992, 8)],
                            our.at[pl.ds(lo + 4992, 8)])
        if ci < 3:
            plsc.subcore_barrier()


def _seg_call(ms, sidx2d, didx2d, zer):
    f = functools.partial(
        pl.kernel, mesh=_mesh(),
        out_type=tuple(
            jax.ShapeDtypeStruct((_N, 128), jnp.float32) for _ in range(4)),
        scratch_types=[
            pltpu.VMEM((_NB, _K), jnp.int32),
            pltpu.VMEM((_NB, _K), jnp.int32),
            pltpu.VMEM((_PBUF,), jnp.int32),
            pltpu.VMEM((_PBUF,), jnp.int32),
            pltpu.VMEM((_B, 128), jnp.float32),
            pltpu.VMEM((_B, 128), jnp.float32),
            pltpu.VMEM_SHARED((_HACC, 128), jnp.float32),
            pltpu.SemaphoreType.DMA,
            pltpu.SemaphoreType.DMA,
        ],
        compiler_params=pltpu.CompilerParams(needs_layout_passes=False),
    )(_seg_body)
    return f(ms[0], ms[1], ms[2], ms[3], sidx2d, didx2d, zer)


def _m1_body(deg_ref, x_ref, w_ref, o0, o1, o2, o3):
    dso = lax.rsqrt(jnp.maximum(deg_ref[:, 0:1], 1.0))
    y = jnp.dot(x_ref[...] * dso, w_ref[...],
                preferred_element_type=jnp.float32)
    for j, o in enumerate((o0, o1, o2, o3)):
        o[...] = y[:, j * 128:(j + 1) * 128]


def _m1_call(degt, x, w1):
    return pl.pallas_call(
        _m1_body,
        grid=(_N // 400,),
        in_specs=[
            pl.BlockSpec((400, 2), lambda i: (i, 0)),
            pl.BlockSpec((400, 256), lambda i: (i, 0)),
            pl.BlockSpec((256, 512), lambda i: (0, 0)),
        ],
        out_specs=[pl.BlockSpec((400, 128), lambda i: (i, 0))
                   for _ in range(4)],
        out_shape=[jax.ShapeDtypeStruct((_N, 128), jnp.float32)
                   for _ in range(4)],
        compiler_params=pltpu.CompilerParams(
            dimension_semantics=("parallel",)),
    )(degt, x, w1)


def _layer_body(deg_ref, a0, a1, a2, a3, w_ref, b_ref, fl_ref,
                o0, o1, o2, o3):
    dsi = lax.rsqrt(jnp.maximum(deg_ref[:, 1:2], 1.0))
    h = jnp.concatenate([a0[...], a1[...], a2[...], a3[...]], axis=1) * dsi
    y = jnp.maximum(jnp.dot(h, w_ref[...],
                            preferred_element_type=jnp.float32) + b_ref[...],
                    0.0)
    dso = lax.rsqrt(jnp.maximum(deg_ref[:, 0:1], 1.0))
    fl = fl_ref[0, 0]
    y = y * (dso * fl + (1.0 - fl))
    for j, o in enumerate((o0, o1, o2, o3)):
        o[...] = y[:, j * 128:(j + 1) * 128]


def _layer_call(degt, aggs, w, b2d, fl):
    return pl.pallas_call(
        _layer_body,
        grid=(_N // 400,),
        in_specs=[pl.BlockSpec((400, 2), lambda i: (i, 0))]
        + [pl.BlockSpec((400, 128), lambda i: (i, 0)) for _ in range(4)]
        + [
            pl.BlockSpec((512, 512), lambda i: (0, 0)),
            pl.BlockSpec((1, 512), lambda i: (0, 0)),
            pl.BlockSpec((1, 1), lambda i: (0, 0)),
        ],
        out_specs=[pl.BlockSpec((400, 128), lambda i: (i, 0))
                   for _ in range(4)],
        out_shape=[jax.ShapeDtypeStruct((_N, 128), jnp.float32)
                   for _ in range(4)],
        compiler_params=pltpu.CompilerParams(
            dimension_semantics=("parallel",)),
    )(degt, aggs[0], aggs[1], aggs[2], aggs[3], w, b2d, fl)


def _pool_body(y0, y1, y2, y3, wfc_ref, bfc_ref, o_ref, sum_ref):
    i = pl.program_id(0)

    @pl.when(i == 0)
    def _():
        sum_ref[...] = jnp.zeros_like(sum_ref)

    yc = jnp.concatenate([y0[...], y1[...], y2[...], y3[...]], axis=1)
    sum_ref[...] += jnp.sum(yc, axis=0, keepdims=True)

    @pl.when(i == _N // 400 - 1)
    def _():
        total = jnp.sum(sum_ref[...] * wfc_ref[...])
        o_ref[...] = jnp.reshape(total / float(_N) + bfc_ref[0, 0], (1, 1))


def _pool_call(ys, wfc2d, bfc2d):
    return pl.pallas_call(
        _pool_body,
        grid=(_N // 400,),
        in_specs=[pl.BlockSpec((400, 128), lambda i: (i, 0))
                  for _ in range(4)]
        + [
            pl.BlockSpec((1, 512), lambda i: (0, 0)),
            pl.BlockSpec((1, 1), lambda i: (0, 0)),
        ],
        out_specs=pl.BlockSpec((1, 1), lambda i: (0, 0)),
        out_shape=jax.ShapeDtypeStruct((1, 1), jnp.float32),
        scratch_shapes=[pltpu.VMEM((1, 512), jnp.float32)],
        compiler_params=pltpu.CompilerParams(
            dimension_semantics=("arbitrary",)),
    )(ys[0], ys[1], ys[2], ys[3], wfc2d, bfc2d)


def kernel(x, edge_index, W1, b1, W2, b2, W3, b3, W_fc, b_fc):
    src = edge_index[0].reshape(_NS, _EPT)
    dst = edge_index[1].reshape(_NS, _EPT)
    pad = ((0, 0), (0, _EPAD - _EPT))
    sidx = jnp.pad(src, pad).reshape(_NS * _NB, _K)
    sidx_t = jnp.pad(src, pad, constant_values=_N).reshape(_NS * _NB, _K)
    didx_t = jnp.pad(dst, pad, constant_values=_N).reshape(_NS * _NB, _K)
    deg_idx = jnp.concatenate([sidx_t, didx_t], axis=0)

    deg = _deg_call(deg_idx)
    degt = jnp.stack([deg[0, 0, :_N], deg[1, 0, :_N]], axis=1)

    wstack = jnp.stack([jnp.eye(512, dtype=jnp.float32), W2, W3])
    bstack = jnp.stack([b1, b2, b3]).reshape(3, 1, 512)
    fstack = jnp.array([1.0, 1.0, 0.0], jnp.float32).reshape(3, 1, 1)
    zer = jnp.zeros((312, 128), jnp.float32)

    ms = tuple(_m1_call(degt, x, W1))

    def _layer(l, ms):
        aggs = _seg_call(ms, sidx, didx_t, zer)
        wl = lax.dynamic_index_in_dim(wstack, l, keepdims=False)
        bl = lax.dynamic_index_in_dim(bstack, l, keepdims=False)
        fl = lax.dynamic_index_in_dim(fstack, l, keepdims=False)
        return tuple(_layer_call(degt, aggs, wl, bl, fl))

    ys = lax.fori_loop(0, 3, _layer, ms)
    out = _pool_call(ys, W_fc.reshape(1, 512), b_fc.reshape(1, 1))
    return out.reshape(())


# speedup vs baseline: 2.2179x; 1.0305x over previous
import functools

import jax
import jax.numpy as jnp
from jax import lax
from jax.experimental import pallas as pl
from jax.experimental.pallas import tpu as pltpu
from jax.experimental.pallas import tpu_sc as plsc

_N = 10000
_E = 160000
_NS = 16
_NC = 2
_EPT = _E // _NS
_K = 128
_NB = 80
_EPAD = _NB * _K
_HN = 5000
_HACC = 5008
_PBUF = 10752
_B = 128
_NH = 10240
_HPT = _NH // _NS


def _mesh():
    return plsc.VectorSubcoreMesh(core_axis_name="c", subcore_axis_name="s")


def _deg_body(idx2d, out_hbm, stage_hbm, sp_hbm, dp_hbm, cnt_hbm,
              didx_v, hist_v, red_v, sum_v, sidx_v, didx2_v, sp_v, dp_v,
              cnt_v):
    cid = lax.axis_index("c")
    sid = lax.axis_index("s")
    one16 = jnp.full((16,), 1.0, jnp.float32)
    zer16 = jnp.zeros((16,), jnp.float32)

    def _fill_zer(i, c):
        hist_v[pl.ds(i * 16, 16)] = zer16
        return c
    lax.fori_loop(0, _NH // 16, _fill_zer, 0)

    pltpu.sync_copy(idx2d.at[pl.ds(cid * (_NS * _NB) + sid * _NB, _NB)],
                    didx_v)

    def _hist(b, c):
        for j in range(_K // 16):
            idx16 = didx_v[b, pl.ds(j * 16, 16)]
            plsc.addupdate_scatter(hist_v, [idx16], one16)
        return c
    lax.fori_loop(0, _NB, _hist, 0)

    pltpu.sync_copy(hist_v, stage_hbm.at[cid * _NS + sid, 0])

    pltpu.sync_copy(idx2d.at[pl.ds(sid * _NB, _NB)], sidx_v)
    pltpu.sync_copy(idx2d.at[pl.ds(_NS * _NB + sid * _NB, _NB)], didx2_v)

    zero16i = jnp.zeros((16,), jnp.int32)
    trash16 = jnp.full((16,), _HN, jnp.int32)

    def _fill(i, c):
        sp_v[pl.ds(i * 16, 16)] = zero16i
        dp_v[pl.ds(i * 16, 16)] = trash16
        return c
    lax.fori_loop(0, _PBUF // 16, _fill, 0)

    lo = cid * _HN

    def _part(b, off):
        for j in range(_K // 16):
            d = didx2_v[b, pl.ds(j * 16, 16)]
            sv = sidx_v[b, pl.ds(j * 16, 16)]
            dl = d - lo
            msk = jnp.logical_and(dl >= 0, dl < _HN)
            plsc.store_compressed(sp_v.at[pl.ds(off, 16)], sv, mask=msk)
            plsc.store_compressed(dp_v.at[pl.ds(off, 16)], dl, mask=msk)
            off = off + plsc.all_reduce_population_count(msk)[0]
        return off
    cnt = lax.fori_loop(0, _NB, _part, 0)
    cnt_v[pl.ds(0, 16)] = zero16i + cnt
    row = cid * _NS + sid
    pltpu.sync_copy(sp_v, sp_hbm.at[row, 0])
    pltpu.sync_copy(dp_v, dp_hbm.at[row, 0])
    pltpu.sync_copy(cnt_v, cnt_hbm.at[row, 0])
    plsc.subcore_barrier()

    pltpu.sync_copy(
        stage_hbm.at[pl.ds(cid * _NS, _NS), 0, pl.ds(sid * _HPT, _HPT)],
        red_v)

    def _red(j, c):
        s = zer16
        for r in range(_NS):
            s = s + red_v[r, pl.ds(j * 16, 16)]
        sum_v[pl.ds(j * 16, 16)] = s
        return c
    lax.fori_loop(0, _HPT // 16, _red, 0)

    pltpu.sync_copy(sum_v, out_hbm.at[cid, 0, pl.ds(sid * _HPT, _HPT)])


def _deg_call(idx2d):
    f = functools.partial(
        pl.kernel, mesh=_mesh(),
        compiler_params=pltpu.CompilerParams(needs_layout_passes=False),
        out_type=(jax.ShapeDtypeStruct((_NC, 1, _NH), jnp.float32),
                  jax.ShapeDtypeStruct((_NC * _NS, 1, _NH), jnp.float32),
                  jax.ShapeDtypeStruct((_NC * _NS, 1, _PBUF), jnp.int32),
                  jax.ShapeDtypeStruct((_NC * _NS, 1, _PBUF), jnp.int32),
                  jax.ShapeDtypeStruct((_NC * _NS, 1, 16), jnp.int32)),
        scratch_types=[
            pltpu.VMEM((_NB, _K), jnp.int32),
            pltpu.VMEM((_NH,), jnp.float32),
            pltpu.VMEM((_NS, _HPT), jnp.float32),
            pltpu.VMEM((_HPT,), jnp.float32),
            pltpu.VMEM((_NB, _K), jnp.int32),
            pltpu.VMEM((_NB, _K), jnp.int32),
            pltpu.VMEM((_PBUF,), jnp.int32),
            pltpu.VMEM((_PBUF,), jnp.int32),
            pltpu.VMEM((16,), jnp.int32),
        ],
    )(_deg_body)
    out = f(idx2d)
    return out[0], out[2], out[3], out[4]


def _seg_body(m0, m1, m2, m3, sp_hbm, dp_hbm, cnt_hbm, zer_hbm,
              o0, o1, o2, o3,
              sp_v, dp_v, cnt_v, r0, r1, r2, r3, acc_sh, sem0, sem1):
    cid = lax.axis_index("c")
    sid = lax.axis_index("s")
    lo = cid * _HN
    row = cid * _NS + sid

    pltpu.sync_copy(sp_hbm.at[row, 0], sp_v)
    pltpu.sync_copy(dp_hbm.at[row, 0], dp_v)
    pltpu.sync_copy(cnt_hbm.at[row, 0], cnt_v)
    cnt = cnt_v[pl.ds(0, 16)][0]
    nb = ((cnt + 2 * _B - 1) // (2 * _B)) * 2

    bufs = (r0, r1, r2, r3)
    sems = (sem0, sem1, sem0, sem1)

    for ci, (mr, our) in enumerate(((m0, o0), (m1, o1), (m2, o2), (m3, o3))):
        pltpu.sync_copy(zer_hbm.at[pl.ds(0, 312)],
                        acc_sh.at[pl.ds(sid * 312, 312)])

        @pl.when(sid == _NS - 1)
        def _():
            pltpu.sync_copy(zer_hbm.at[pl.ds(0, 16)],
                            acc_sh.at[pl.ds(4992, 16)])
        plsc.subcore_barrier()

        def _gather(b, buf, sem, mr=mr):
            pltpu.async_copy(mr.at[sp_v.at[pl.ds(b * _B, _B)]], buf, sem)

        def _wait(buf, sem, mr=mr):
            pltpu.make_async_copy(mr.at[pl.ds(0, _B)], buf, sem).wait()

        for i in range(4):
            @pl.when(i < nb)
            def _(i=i):
                _gather(i, bufs[i], sems[i])

        def _quad(p, c):
            b0 = 4 * p
            for i in range(4):
                b = b0 + i

                @pl.when(b < nb)
                def _(b=b, i=i):
                    _wait(bufs[i], sems[i])
                    pltpu.sync_copy(
                        bufs[i], acc_sh.at[dp_v.at[pl.ds(b * _B, _B)]],
                        add=True)

                    @pl.when(b + 4 < nb)
                    def _():
                        _gather(b + 4, bufs[i], sems[i])
            return c
        lax.fori_loop(0, (nb + 3) // 4, _quad, 0)

        plsc.subcore_barrier()
        pltpu.sync_copy(acc_sh.at[pl.ds(sid * 312, 312)],
                        our.at[pl.ds(lo + sid * 312, 312)])

        @pl.when(sid == _NS - 1)
        def _():
            pltpu.sync_copy(acc_sh.at[pl.ds(4992, 8)],
                            our.at[pl.ds(lo + 4992, 8)])
        if ci < 3:
            plsc.subcore_barrier()


def _seg_call(ms, sp, dp, cnts, zer):
    f = functools.partial(
        pl.kernel, mesh=_mesh(),
        out_type=tuple(
            jax.ShapeDtypeStruct((_N, 128), jnp.float32) for _ in range(4)),
        scratch_types=[
            pltpu.VMEM((_PBUF,), jnp.int32),
            pltpu.VMEM((_PBUF,), jnp.int32),
            pltpu.VMEM((16,), jnp.int32),
            pltpu.VMEM((_B, 128), jnp.float32),
            pltpu.VMEM((_B, 128), jnp.float32),
            pltpu.VMEM((_B, 128), jnp.float32),
            pltpu.VMEM((_B, 128), jnp.float32),
            pltpu.VMEM_SHARED((_HACC, 128), jnp.float32),
            pltpu.SemaphoreType.DMA,
            pltpu.SemaphoreType.DMA,
        ],
    )(_seg_body)
    return f(ms[0], ms[1], ms[2], ms[3], sp, dp, cnts, zer)


def _m1_body(deg_ref, x_ref, w_ref, o0, o1, o2, o3):
    dso = lax.rsqrt(jnp.maximum(deg_ref[:, 0:1], 1.0))
    y = jnp.dot(x_ref[...] * dso, w_ref[...],
                preferred_element_type=jnp.float32)
    for j, o in enumerate((o0, o1, o2, o3)):
        o[...] = y[:, j * 128:(j + 1) * 128]


def _m1_call(degt, x, w1):
    return pl.pallas_call(
        _m1_body,
        grid=(_N // 400,),
        in_specs=[
            pl.BlockSpec((400, 2), lambda i: (i, 0)),
            pl.BlockSpec((400, 256), lambda i: (i, 0)),
            pl.BlockSpec((256, 512), lambda i: (0, 0)),
        ],
        out_specs=[pl.BlockSpec((400, 128), lambda i: (i, 0))
                   for _ in range(4)],
        out_shape=[jax.ShapeDtypeStruct((_N, 128), jnp.float32)
                   for _ in range(4)],
        compiler_params=pltpu.CompilerParams(
            dimension_semantics=("parallel",)),
    )(degt, x, w1)


def _layer_body(deg_ref, a0, a1, a2, a3, w_ref, b_ref, fl_ref,
                o0, o1, o2, o3):
    dsi = lax.rsqrt(jnp.maximum(deg_ref[:, 1:2], 1.0))
    h = jnp.concatenate([a0[...], a1[...], a2[...], a3[...]], axis=1) * dsi
    y = jnp.maximum(jnp.dot(h, w_ref[...],
                            preferred_element_type=jnp.float32) + b_ref[...],
                    0.0)
    dso = lax.rsqrt(jnp.maximum(deg_ref[:, 0:1], 1.0))
    fl = fl_ref[0, 0]
    y = y * (dso * fl + (1.0 - fl))
    for j, o in enumerate((o0, o1, o2, o3)):
        o[...] = y[:, j * 128:(j + 1) * 128]


def _layer_call(degt, aggs, w, b2d, fl):
    return pl.pallas_call(
        _layer_body,
        grid=(_N // 400,),
        in_specs=[pl.BlockSpec((400, 2), lambda i: (i, 0))]
        + [pl.BlockSpec((400, 128), lambda i: (i, 0)) for _ in range(4)]
        + [
            pl.BlockSpec((512, 512), lambda i: (0, 0)),
            pl.BlockSpec((1, 512), lambda i: (0, 0)),
            pl.BlockSpec((1, 1), lambda i: (0, 0)),
        ],
        out_specs=[pl.BlockSpec((400, 128), lambda i: (i, 0))
                   for _ in range(4)],
        out_shape=[jax.ShapeDtypeStruct((_N, 128), jnp.float32)
                   for _ in range(4)],
        compiler_params=pltpu.CompilerParams(
            dimension_semantics=("parallel",)),
    )(degt, aggs[0], aggs[1], aggs[2], aggs[3], w, b2d, fl)


def _pool_body(y0, y1, y2, y3, wfc_ref, bfc_ref, o_ref, sum_ref):
    i = pl.program_id(0)

    @pl.when(i == 0)
    def _():
        sum_ref[...] = jnp.zeros_like(sum_ref)

    yc = jnp.concatenate([y0[...], y1[...], y2[...], y3[...]], axis=1)
    sum_ref[...] += jnp.sum(yc, axis=0, keepdims=True)

    @pl.when(i == _N // 400 - 1)
    def _():
        total = jnp.sum(sum_ref[...] * wfc_ref[...])
        o_ref[...] = jnp.reshape(total / float(_N) + bfc_ref[0, 0], (1, 1))


def _pool_call(ys, wfc2d, bfc2d):
    return pl.pallas_call(
        _pool_body,
        grid=(_N // 400,),
        in_specs=[pl.BlockSpec((400, 128), lambda i: (i, 0))
                  for _ in range(4)]
        + [
            pl.BlockSpec((1, 512), lambda i: (0, 0)),
            pl.BlockSpec((1, 1), lambda i: (0, 0)),
        ],
        out_specs=pl.BlockSpec((1, 1), lambda i: (0, 0)),
        out_shape=jax.ShapeDtypeStruct((1, 1), jnp.float32),
        scratch_shapes=[pltpu.VMEM((1, 512), jnp.float32)],
        compiler_params=pltpu.CompilerParams(
            dimension_semantics=("arbitrary",)),
    )(ys[0], ys[1], ys[2], ys[3], wfc2d, bfc2d)


def kernel(x, edge_index, W1, b1, W2, b2, W3, b3, W_fc, b_fc):
    src = edge_index[0].reshape(_NS, _EPT)
    dst = edge_index[1].reshape(_NS, _EPT)
    pad = ((0, 0), (0, _EPAD - _EPT))
    sidx = jnp.pad(src, pad).reshape(_NS * _NB, _K)
    sidx_t = jnp.pad(src, pad, constant_values=_N).reshape(_NS * _NB, _K)
    didx_t = jnp.pad(dst, pad, constant_values=_N).reshape(_NS * _NB, _K)
    deg_idx = jnp.concatenate([sidx_t, didx_t], axis=0)

    deg, sp, dp, cnts = _deg_call(deg_idx)
    degt = jnp.stack([deg[0, 0, :_N], deg[1, 0, :_N]], axis=1)

    wstack = jnp.stack([jnp.eye(512, dtype=jnp.float32), W2, W3])
    bstack = jnp.stack([b1, b2, b3]).reshape(3, 1, 512)
    fstack = jnp.array([1.0, 1.0, 0.0], jnp.float32).reshape(3, 1, 1)
    zer = jnp.zeros((312, 128), jnp.float32)

    ms = tuple(_m1_call(degt, x, W1))

    def _layer(l, ms):
        aggs = _seg_call(ms, sp, dp, cnts, zer)
        wl = lax.dynamic_index_in_dim(wstack, l, keepdims=False)
        bl = lax.dynamic_index_in_dim(bstack, l, keepdims=False)
        fl = lax.dynamic_index_in_dim(fstack, l, keepdims=False)
        return tuple(_layer_call(degt, aggs, wl, bl, fl))

    ys = lax.fori_loop(0, 3, _layer, ms)
    out = _pool_call(ys, W_fc.reshape(1, 512), b_fc.reshape(1, 1))
    return out.reshape(())
